# baseline (speedup 1.0000x reference)
import functools

import jax
import jax.numpy as jnp
from jax import lax
from jax.experimental import pallas as pl
from jax.experimental.pallas import tpu as pltpu
from jax.experimental.pallas import tpu_sc as plsc

_NUM_FIXED = 900000
_EMBED_DIM = 64
_LANES = 16

_NC = 2
_NS = 16
_NW = _NC * _NS

_CHUNK = 512
_BLK = 128
_NBLK = _CHUNK // _BLK


def _sc_body(n_rows, per_w, idx_hbm, fixed_hbm, train_hbm, out_hbm,
             idx_v, *scratch):
    fidx = scratch[0:_NBLK]
    tidx = scratch[_NBLK:2 * _NBLK]
    pos = scratch[2 * _NBLK:3 * _NBLK]
    rows_v, trows_v, gsem, ssem = scratch[3 * _NBLK:]

    wid = lax.axis_index("s") * _NC + lax.axis_index("c")
    base = wid * per_w
    dummy_row = n_rows + wid
    lane = lax.iota(jnp.int32, _LANES)

    def chunk_body(ci, carry):
        cbase = base + ci * _CHUNK
        pltpu.sync_copy(idx_hbm.at[pl.ds(cbase, _CHUNK)], idx_v)

        for g in range(_CHUNK // _LANES):
            j, col = g // (_BLK // _LANES), (g % (_BLK // _LANES)) * _LANES
            v = idx_v[pl.ds(g * _LANES, _LANES)]
            m = v < _NUM_FIXED
            fidx[j][pl.ds(col, _LANES)] = jnp.where(m, v, 0)
            tidx[j][pl.ds(col, _LANES)] = jnp.where(m, 0, v - _NUM_FIXED)
            pos[j][pl.ds(col, _LANES)] = jnp.where(
                m, dummy_row, cbase + g * _LANES + lane)

        cps = [
            pltpu.async_copy(fixed_hbm.at[fidx[j]],
                             rows_v.at[pl.ds(j * _BLK, _BLK)], gsem)
            for j in range(_NBLK)
        ]
        for cp in cps:
            cp.wait()
        pltpu.sync_copy(rows_v, out_hbm.at[pl.ds(cbase, _CHUNK)])

        cps = [
            pltpu.async_copy(train_hbm.at[tidx[j]],
                             trows_v.at[pl.ds(j * _BLK, _BLK)], gsem)
            for j in range(_NBLK)
        ]
        for cp in cps:
            cp.wait()
        cps = [
            pltpu.async_copy(trows_v.at[pl.ds(j * _BLK, _BLK)],
                             out_hbm.at[pos[j]], ssem)
            for j in range(_NBLK)
        ]
        for cp in cps:
            cp.wait()
        return carry

    lax.fori_loop(0, per_w // _CHUNK, chunk_body, 0)


@jax.jit
def _embed_lookup(idx_flat, fixed_weights, trainable_weight):
    n_rows = idx_flat.shape[0]
    per_w = n_rows // _NW
    mesh = plsc.VectorSubcoreMesh(core_axis_name="c", subcore_axis_name="s",
                                  num_cores=_NC, num_subcores=_NS)
    body = functools.partial(_sc_body, n_rows, per_w)
    out = pl.kernel(
        body,
        out_type=jax.ShapeDtypeStruct((n_rows + _NW, _EMBED_DIM),
                                      jnp.float32),
        mesh=mesh,
        compiler_params=pltpu.CompilerParams(use_tc_tiling_on_sc=False),
        scratch_types=(
            [pltpu.VMEM((_CHUNK,), jnp.int32)]
            + [pltpu.VMEM((_BLK,), jnp.int32) for _ in range(3 * _NBLK)]
            + [
                pltpu.VMEM((_CHUNK, _EMBED_DIM), jnp.float32),
                pltpu.VMEM((_CHUNK, _EMBED_DIM), jnp.float32),
                pltpu.SemaphoreType.DMA,
                pltpu.SemaphoreType.DMA,
            ]
        ),
    )(idx_flat, fixed_weights, trainable_weight)
    return out[:n_rows]


def kernel(inp, fixed_weights, trainable_weight):
    b, s = inp.shape
    idx_flat = inp.reshape(-1).astype(jnp.int32)
    out = _embed_lookup(idx_flat, fixed_weights, trainable_weight)
    return out.reshape(b, s, _EMBED_DIM)


# speedup vs baseline: 10.3159x; 10.3159x over previous
import functools

import jax
import jax.numpy as jnp
from jax import lax
from jax.experimental import pallas as pl
from jax.experimental.pallas import tpu as pltpu
from jax.experimental.pallas import tpu_sc as plsc

_NUM_FIXED = 900000
_EMBED_DIM = 64
_LANES = 16

_NC = 2
_NS = 16
_NW = _NC * _NS

_CHUNK = 512
_BLK = 128
_NBLK = _CHUNK // _BLK


def _sc_body(n_rows, per_w, idx_hbm, fixed_hbm, train_hbm, out_hbm,
             idx_v, *scratch):
    fidx = scratch[0:_NBLK]
    tidx = scratch[_NBLK:2 * _NBLK]
    pos = scratch[2 * _NBLK:3 * _NBLK]
    rows_v, trows_v, gsem, ssem = scratch[3 * _NBLK:]

    wid = lax.axis_index("s") * _NC + lax.axis_index("c")
    base = wid * per_w
    pad_base = n_rows + wid * _CHUNK
    lane = lax.iota(jnp.int32, _LANES)

    def chunk_body(ci, carry):
        cbase = base + ci * _CHUNK
        pltpu.sync_copy(idx_hbm.at[pl.ds(cbase, _CHUNK)], idx_v)

        for g in range(_CHUNK // _LANES):
            j, col = g // (_BLK // _LANES), (g % (_BLK // _LANES)) * _LANES
            v = idx_v[pl.ds(g * _LANES, _LANES)]
            co = lane + (g * _LANES)
            m = v < _NUM_FIXED
            fidx[j][pl.ds(col, _LANES)] = jnp.where(m, v, co)
            tidx[j][pl.ds(col, _LANES)] = jnp.where(m, co, v - _NUM_FIXED)
            pos[j][pl.ds(col, _LANES)] = jnp.where(
                m, pad_base + co, cbase + co)

        cps = [
            pltpu.async_copy(fixed_hbm.at[fidx[j]],
                             rows_v.at[pl.ds(j * _BLK, _BLK)], gsem)
            for j in range(_NBLK)
        ]
        for cp in cps:
            cp.wait()
        pltpu.sync_copy(rows_v, out_hbm.at[pl.ds(cbase, _CHUNK)])

        cps = [
            pltpu.async_copy(train_hbm.at[tidx[j]],
                             trows_v.at[pl.ds(j * _BLK, _BLK)], gsem)
            for j in range(_NBLK)
        ]
        for cp in cps:
            cp.wait()
        cps = [
            pltpu.async_copy(trows_v.at[pl.ds(j * _BLK, _BLK)],
                             out_hbm.at[pos[j]], ssem)
            for j in range(_NBLK)
        ]
        for cp in cps:
            cp.wait()
        return carry

    lax.fori_loop(0, per_w // _CHUNK, chunk_body, 0)


@jax.jit
def _embed_lookup(idx_flat, fixed_weights, trainable_weight):
    n_rows = idx_flat.shape[0]
    per_w = n_rows // _NW
    mesh = plsc.VectorSubcoreMesh(core_axis_name="c", subcore_axis_name="s",
                                  num_cores=_NC, num_subcores=_NS)
    body = functools.partial(_sc_body, n_rows, per_w)
    out = pl.kernel(
        body,
        out_type=jax.ShapeDtypeStruct((n_rows + _NW * _CHUNK, _EMBED_DIM),
                                      jnp.float32),
        mesh=mesh,
        compiler_params=pltpu.CompilerParams(use_tc_tiling_on_sc=False),
        scratch_types=(
            [pltpu.VMEM((_CHUNK,), jnp.int32)]
            + [pltpu.VMEM((_BLK,), jnp.int32) for _ in range(3 * _NBLK)]
            + [
                pltpu.VMEM((_CHUNK, _EMBED_DIM), jnp.float32),
                pltpu.VMEM((_CHUNK, _EMBED_DIM), jnp.float32),
                pltpu.SemaphoreType.DMA,
                pltpu.SemaphoreType.DMA,
            ]
        ),
    )(idx_flat, fixed_weights, trainable_weight)
    return out[:n_rows]


def kernel(inp, fixed_weights, trainable_weight):
    b, s = inp.shape
    idx_flat = inp.reshape(-1).astype(jnp.int32)
    out = _embed_lookup(idx_flat, fixed_weights, trainable_weight)
    return out.reshape(b, s, _EMBED_DIM)
